# Initial kernel scaffold; baseline (speedup 1.0000x reference)
import jax
import jax.numpy as jnp
from jax.experimental import pallas as pl


def kernel(obj_vecs, pred_vecs, triples, params):
    raise NotImplementedError("write your pallas kernel here")


# speedup vs baseline: 3.2063x; 3.2063x over previous
import functools

import jax
import jax.numpy as jnp
from jax import lax
from jax.experimental import pallas as pl
from jax.experimental.pallas import tpu as pltpu
from jax.experimental.pallas import tpu_sc as plsc

B = 16
OBJ = 6250
NC, NS = 2, 16
NW = NC * NS
CH = 392
OBJP = NS * CH
NP = B * OBJP
GPW = NP // NW
BLK = 2048


def _gather_call(D):
    mesh = plsc.VectorSubcoreMesh(core_axis_name="c", subcore_axis_name="s")
    sds = jax.ShapeDtypeStruct((NP, D), jnp.float32)

    @functools.partial(
        pl.kernel,
        out_type=(sds, sds),
        mesh=mesh,
        scratch_types=[
            pltpu.VMEM((CH,), jnp.int32),
            pltpu.VMEM((CH, D), jnp.float32),
            pltpu.SemaphoreType.DMA,
        ],
    )
    def gather_k(table, gs, go, curs, curo, idx, rows, sem):
        c = lax.axis_index("c")
        t = lax.axis_index("s")
        start = (c * NS + t) * GPW
        for idxarr, outref in ((gs, curs), (go, curo)):
            for j in range(GPW // CH):
                off = start + j * CH
                pltpu.sync_copy(idxarr.at[pl.ds(off, CH)], idx)
                pltpu.async_copy(table.at[idx], rows, sem).wait()
                pltpu.sync_copy(rows, outref.at[pl.ds(off, CH)])

    return gather_k


def _scatter_call(W):
    mesh = plsc.VectorSubcoreMesh(core_axis_name="c", subcore_axis_name="s")

    npass = W // 128

    @functools.partial(
        pl.kernel,
        out_type=jax.ShapeDtypeStruct((NP, W), jnp.float32),
        mesh=mesh,
        scratch_types=[
            pltpu.VMEM((CH, 128), jnp.float32),
            pltpu.VMEM((56, 128), jnp.float32),
            pltpu.VMEM((CH,), jnp.int32),
            pltpu.VMEM_SHARED((OBJP, 128), jnp.float32),
        ],
    )
    def scatter_k(srcs, srco, sidx, oidx, out, rows, zb, idx, table):
        c = lax.axis_index("c")
        t = lax.axis_index("s")
        zv = jnp.zeros((16,), jnp.float32)

        def zrow(r, carry):
            for cc in range(128 // 16):
                zb[r, pl.ds(cc * 16, 16)] = zv
            return carry

        lax.fori_loop(0, 56, zrow, 0)

        for j in range(B // NC):
            b = c * (B // NC) + j
            base = b * OBJP + t * CH
            for cp in range(npass):
                col = cp * 128
                for k in range(CH // 56):
                    pltpu.sync_copy(zb, table.at[pl.ds(t * CH + k * 56, 56)])
                plsc.subcore_barrier()
                for src, idxarr in ((srcs, sidx), (srco, oidx)):
                    pltpu.sync_copy(idxarr.at[pl.ds(base, CH)], idx)
                    pltpu.sync_copy(
                        src.at[pl.ds(base, CH), pl.ds(col, 128)], rows)
                    pltpu.sync_copy(rows, table.at[idx], add=True)
                plsc.subcore_barrier()
                pltpu.sync_copy(table.at[pl.ds(t * CH, CH)],
                                out.at[pl.ds(base, CH), pl.ds(col, 128)])
                plsc.subcore_barrier()

    return scatter_k


def _triple_mlp_call(Din, H, Dout):
    grid = (NP // BLK,)

    def body(s, p, o, ws, wp, wo, b1, w2s, b2s, w2p, b2p, w2o, b2o,
             nsa, npv, noa):
        m = pl.program_id(0)
        h = s[...] @ ws[...] + p[...] @ wp[...] + o[...] @ wo[...] + b1[...]
        h = jnp.maximum(h, 0.0)
        rloc = (lax.broadcasted_iota(jnp.int32, (BLK, 1), 0) + m * BLK) % OBJP
        keep = (rloc < OBJ).astype(jnp.float32)
        nsa[...] = jnp.maximum(h @ w2s[...] + b2s[...], 0.0) * keep
        npv[...] = jnp.maximum(h @ w2p[...] + b2p[...], 0.0) * keep
        noa[...] = jnp.maximum(h @ w2o[...] + b2o[...], 0.0) * keep

    def full(shape):
        return pl.BlockSpec(shape, lambda m: (0, 0))

    rowblk = lambda d: pl.BlockSpec((BLK, d), lambda m: (m, 0))
    return pl.pallas_call(
        body,
        grid=grid,
        in_specs=[
            rowblk(Din), rowblk(Din), rowblk(Din),
            full((Din, H)), full((Din, H)), full((Din, H)), full((1, H)),
            full((H, H)), full((1, H)),
            full((H, Dout)), full((1, Dout)),
            full((H, H)), full((1, H)),
        ],
        out_specs=[rowblk(H), rowblk(Dout), rowblk(H)],
        out_shape=[
            jax.ShapeDtypeStruct((NP, H), jnp.float32),
            jax.ShapeDtypeStruct((NP, Dout), jnp.float32),
            jax.ShapeDtypeStruct((NP, H), jnp.float32),
        ],
    )


def _node_mlp_call(H, Dout):
    grid = (NP // BLK,)

    def body(pa, cnt, w1, b1, w2, b2, outref):
        rcp = 1.0 / jnp.maximum(cnt[...][:, :1], 1.0)
        pooled = pa[...] * rcp
        h2 = jnp.maximum(pooled @ w1[...] + b1[...], 0.0)
        outref[...] = jnp.maximum(h2 @ w2[...] + b2[...], 0.0)

    def full(shape):
        return pl.BlockSpec(shape, lambda m: (0, 0))

    return pl.pallas_call(
        body,
        grid=grid,
        in_specs=[
            pl.BlockSpec((BLK, H), lambda m: (m, 0)),
            pl.BlockSpec((BLK, 8), lambda m: (m, 0)),
            full((H, H)), full((1, H)), full((H, Dout)), full((1, Dout)),
        ],
        out_specs=pl.BlockSpec((BLK, Dout), lambda m: (m, 0)),
        out_shape=jax.ShapeDtypeStruct((NP, Dout), jnp.float32),
    )


def kernel(obj_vecs, pred_vecs, triples, params):
    padrows = ((0, 0), (0, OBJP - OBJ), (0, 0))
    ov = jnp.pad(obj_vecs, padrows).reshape(NP, 128)
    pv = jnp.pad(pred_vecs, padrows).reshape(NP, 128)
    s_loc = triples[:, :, 0]
    o_loc = triples[:, :, 1]
    padidx = ((0, 0), (0, OBJP - OBJ))
    sidx = jnp.pad(s_loc, padidx).reshape(NP)
    oidx = jnp.pad(o_loc, padidx).reshape(NP)
    goff = (jnp.arange(B, dtype=triples.dtype) * OBJP)[:, None]
    gs = sidx + jnp.broadcast_to(goff, (B, OBJP)).reshape(NP)
    go = oidx + jnp.broadcast_to(goff, (B, OBJP)).reshape(NP)

    keep = ((jnp.arange(NP, dtype=jnp.int32) % OBJP) < OBJ)
    ones = jnp.broadcast_to(keep.astype(jnp.float32)[:, None], (NP, 128))
    counts = _scatter_call(128)(ones, ones, sidx, oidx)
    counts8 = lax.slice(counts, (0, 0), (NP, 8))

    for li, p in enumerate(params):
        Din = 128 if li == 0 else 256
        H = p['n1w1'].shape[1]
        Dout = p['n2w2'].shape[1]
        ws = p['n1w1'][:Din]
        wp = p['n1w1'][Din:2 * Din]
        wo = p['n1w1'][2 * Din:]
        b1 = p['n1b1'].reshape(1, H)
        w2s = p['n1w2'][:, :H]
        w2p = p['n1w2'][:, H:H + Dout]
        w2o = p['n1w2'][:, H + Dout:]
        b2s = p['n1b2'][:H].reshape(1, H)
        b2p = p['n1b2'][H:H + Dout].reshape(1, Dout)
        b2o = p['n1b2'][H + Dout:].reshape(1, H)

        curs, curo = _gather_call(Din)(ov, gs, go)
        nsa, npv, noa = _triple_mlp_call(Din, H, Dout)(
            curs, pv, curo, ws, wp, wo, b1, w2s, b2s, w2p, b2p, w2o, b2o)
        pooled = _scatter_call(H)(nsa, noa, sidx, oidx)
        ov = _node_mlp_call(H, Dout)(
            pooled, counts8, p['n2w1'], p['n2b1'].reshape(1, H),
            p['n2w2'], p['n2b2'].reshape(1, Dout))
        pv = npv
    return ov.reshape(B, OBJP, -1)[:, :OBJ, :]


# speedup vs baseline: 3.9498x; 1.2319x over previous
import functools

import jax
import jax.numpy as jnp
from jax import lax
from jax.experimental import pallas as pl
from jax.experimental.pallas import tpu as pltpu
from jax.experimental.pallas import tpu_sc as plsc

B = 16
OBJ = 6250
NC, NS = 2, 16
NW = NC * NS
CH = 392
OBJP = NS * CH
NHALF = 2
BH = B // NHALF
NPH = BH * OBJP
BLK = 1792
CHG = 224


def _gather_call(D):
    mesh = plsc.VectorSubcoreMesh(core_axis_name="c", subcore_axis_name="s")
    sds = jax.ShapeDtypeStruct((NPH, D), jnp.float32)
    gpw = NPH // NW
    nchg = gpw // CHG

    @functools.partial(
        pl.kernel,
        out_type=(sds, sds),
        mesh=mesh,
        scratch_types=[
            pltpu.VMEM((CHG,), jnp.int32),
            pltpu.VMEM((CHG,), jnp.int32),
            pltpu.VMEM((CHG, D), jnp.float32),
            pltpu.VMEM((CHG, D), jnp.float32),
            pltpu.SemaphoreType.DMA,
            pltpu.SemaphoreType.DMA,
            pltpu.SemaphoreType.DMA,
            pltpu.SemaphoreType.DMA,
            pltpu.SemaphoreType.DMA,
        ],
    )
    def gather_k(table, gs, go, curs, curo,
                 idx0, idx1, rows0, rows1, gsem, is0, is1, ws0, ws1):
        c = lax.axis_index("c")
        t = lax.axis_index("s")
        start = (c * NS + t) * gpw
        idxb = (idx0, idx1)
        rowsb = (rows0, rows1)
        isem = (is0, is1)
        wsem = (ws0, ws1)
        tasks = []
        for idxarr, outref in ((gs, curs), (go, curo)):
            for j in range(nchg):
                tasks.append((idxarr, outref, start + j * CHG))
        nt = len(tasks)
        pend_idx = {}
        pend_out = {}

        def start_idx(k):
            idxarr, _, off = tasks[k]
            bi = k % 2
            pend_idx[k] = pltpu.async_copy(
                idxarr.at[pl.ds(off, CHG)], idxb[bi], isem[bi])

        start_idx(0)
        start_idx(1)
        for k in range(nt):
            bi = k % 2
            _, outref, off = tasks[k]
            if k >= 2:
                pend_out[k - 2].wait()
            pend_idx[k].wait()
            gd = pltpu.async_copy(table.at[idxb[bi]], rowsb[bi], gsem)
            if k + 2 < nt:
                start_idx(k + 2)
            gd.wait()
            pend_out[k] = pltpu.async_copy(
                rowsb[bi], outref.at[pl.ds(off, CHG)], wsem[bi])
        pend_out[nt - 2].wait()
        pend_out[nt - 1].wait()

    return gather_k


def _scatter_call(W, with_counts=False):
    mesh = plsc.VectorSubcoreMesh(core_axis_name="c", subcore_axis_name="s")

    npass = W // 128
    pooled_t = jax.ShapeDtypeStruct((NPH, W), jnp.float32)
    out_type = ((pooled_t, jax.ShapeDtypeStruct((NPH, 128), jnp.float32))
                if with_counts else pooled_t)
    SA = 200
    SB = CH - SA
    scratch = [
        pltpu.VMEM((SA, 128), jnp.float32),
        pltpu.VMEM((SB, 128), jnp.float32),
        pltpu.VMEM((56, 128), jnp.float32),
        pltpu.VMEM((SA,), jnp.int32),
        pltpu.VMEM((SB,), jnp.int32),
        pltpu.VMEM((SA,), jnp.int32),
        pltpu.VMEM((SB,), jnp.int32),
        pltpu.VMEM_SHARED((OBJP, 128), jnp.float32),
        pltpu.SemaphoreType.DMA,
        pltpu.SemaphoreType.DMA,
    ]

    @functools.partial(
        pl.kernel, out_type=out_type, mesh=mesh, scratch_types=scratch)
    def scatter_k(srcs, srco, sidx, oidx, *rest):
        if with_counts:
            (out, cnt, rows0, rows1, zb, idxsa, idxsb, idxoa, idxob, table,
             ls0, ls1) = rest
        else:
            (out, rows0, rows1, zb, idxsa, idxsb, idxoa, idxob, table,
             ls0, ls1) = rest
            cnt = None
        c = lax.axis_index("c")
        t = lax.axis_index("s")
        rowsb = (rows0, rows1)
        lsem = (ls0, ls1)
        zv = jnp.zeros((16,), jnp.float32)

        def fill(buf, lo, hi, vec):
            def frow(r, carry):
                for cc in range(128 // 16):
                    buf[r, pl.ds(cc * 16, 16)] = vec
                return carry
            lax.fori_loop(lo, hi, frow, 0)

        fill(zb, 0, 56, zv)

        def zero_table():
            for z in range(CH // 56):
                pltpu.sync_copy(zb, table.at[pl.ds(t * CH + z * 56, 56)])

        zero_table()

        nb = BH // NC
        loads = []
        idxrefs = []
        for j in range(nb):
            base = (c * nb + j) * OBJP + t * CH
            for cp in range(npass):
                loads.append((srcs, base, cp * 128, 0))
                loads.append((srcs, base + SA, cp * 128, 1))
                loads.append((srco, base, cp * 128, 0))
                loads.append((srco, base + SA, cp * 128, 1))
                idxrefs.extend([idxsa, idxsb, idxoa, idxob])

        def start_load(k):
            src, base, col, bi = loads[k]
            n = SA if bi == 0 else SB
            return pltpu.async_copy(
                src.at[pl.ds(base, n), pl.ds(col, 128)], rowsb[bi], lsem[bi])

        def load_idx(base):
            pltpu.sync_copy(sidx.at[pl.ds(base, SA)], idxsa)
            pltpu.sync_copy(sidx.at[pl.ds(base + SA, SB)], idxsb)
            pltpu.sync_copy(oidx.at[pl.ds(base, SA)], idxoa)
            pltpu.sync_copy(oidx.at[pl.ds(base + SA, SB)], idxob)

        pend = {0: start_load(0), 1: start_load(1)}
        k = 0
        for j in range(nb):
            base = (c * nb + j) * OBJP + t * CH
            load_idx(base)
            for cp in range(npass):
                col = cp * 128
                for _q in range(4):
                    pend.pop(k).wait()
                    pltpu.sync_copy(rowsb[k % 2], table.at[idxrefs[k]],
                                    add=True)
                    if k + 2 < len(loads):
                        pend[k + 2] = start_load(k + 2)
                    k += 1
                plsc.subcore_barrier()
                pltpu.sync_copy(table.at[pl.ds(t * CH, CH)],
                                out.at[pl.ds(base, CH), pl.ds(col, 128)])
                if k < len(loads) or with_counts:
                    zero_table()
                plsc.subcore_barrier()

        if with_counts:
            one = jnp.ones((16,), jnp.float32)
            fill(rows0, 0, SA, one)
            fill(rows1, 0, SB, one)

            @pl.when(t == NS - 1)
            def _():
                fill(rows1, OBJ - (NS - 1) * CH - SA, SB, zv)

            for j in range(nb):
                base = (c * nb + j) * OBJP + t * CH
                load_idx(base)
                pltpu.sync_copy(rows0, table.at[idxsa], add=True)
                pltpu.sync_copy(rows1, table.at[idxsb], add=True)
                pltpu.sync_copy(rows0, table.at[idxoa], add=True)
                pltpu.sync_copy(rows1, table.at[idxob], add=True)
                plsc.subcore_barrier()
                pltpu.sync_copy(table.at[pl.ds(t * CH, CH)],
                                cnt.at[pl.ds(base, CH)])
                if j + 1 < nb:
                    zero_table()
                plsc.subcore_barrier()

    return scatter_k


def _triple_mlp_call(Din, H, Dout):
    grid = (NPH // BLK,)

    def body(s, p, o, ws, wp, wo, b1, w2s, b2s, w2p, b2p, w2o, b2o,
             nsa, npv, noa):
        m = pl.program_id(0)
        h = s[...] @ ws[...] + p[...] @ wp[...] + o[...] @ wo[...] + b1[...]
        h = jnp.maximum(h, 0.0)
        rloc = (lax.broadcasted_iota(jnp.int32, (BLK, 1), 0) + m * BLK) % OBJP
        keep = (rloc < OBJ).astype(jnp.float32)
        nsa[...] = jnp.maximum(h @ w2s[...] + b2s[...], 0.0) * keep
        npv[...] = jnp.maximum(h @ w2p[...] + b2p[...], 0.0) * keep
        noa[...] = jnp.maximum(h @ w2o[...] + b2o[...], 0.0) * keep

    def full(shape):
        return pl.BlockSpec(shape, lambda m: (0, 0))

    rowblk = lambda d: pl.BlockSpec((BLK, d), lambda m: (m, 0))
    return pl.pallas_call(
        body,
        grid=grid,
        in_specs=[
            rowblk(Din), rowblk(Din), rowblk(Din),
            full((Din, H)), full((Din, H)), full((Din, H)), full((1, H)),
            full((H, H)), full((1, H)),
            full((H, Dout)), full((1, Dout)),
            full((H, H)), full((1, H)),
        ],
        out_specs=[rowblk(H), rowblk(Dout), rowblk(H)],
        out_shape=[
            jax.ShapeDtypeStruct((NPH, H), jnp.float32),
            jax.ShapeDtypeStruct((NPH, Dout), jnp.float32),
            jax.ShapeDtypeStruct((NPH, H), jnp.float32),
        ],
    )


def _node_mlp_call(H, Dout):
    grid = (NPH // BLK,)

    def body(pa, cnt, w1, b1, w2, b2, outref):
        rcp = 1.0 / jnp.maximum(cnt[...][:, :1], 1.0)
        pooled = pa[...] * rcp
        h2 = jnp.maximum(pooled @ w1[...] + b1[...], 0.0)
        outref[...] = jnp.maximum(h2 @ w2[...] + b2[...], 0.0)

    def full(shape):
        return pl.BlockSpec(shape, lambda m: (0, 0))

    return pl.pallas_call(
        body,
        grid=grid,
        in_specs=[
            pl.BlockSpec((BLK, H), lambda m: (m, 0)),
            pl.BlockSpec((BLK, 8), lambda m: (m, 0)),
            full((H, H)), full((1, H)), full((H, Dout)), full((1, Dout)),
        ],
        out_specs=pl.BlockSpec((BLK, Dout), lambda m: (m, 0)),
        out_shape=jax.ShapeDtypeStruct((NPH, Dout), jnp.float32),
    )


def kernel(obj_vecs, pred_vecs, triples, params):
    padrows = ((0, 0), (0, OBJP - OBJ), (0, 0))
    padidx = ((0, 0), (0, OBJP - OBJ))
    goff = (jnp.arange(BH, dtype=triples.dtype) * OBJP)[:, None]

    ov, pv, sidx, oidx, gs, go, counts8 = [], [], [], [], [], [], []
    for hf in range(NHALF):
        sl = slice(hf * BH, (hf + 1) * BH)
        ov.append(jnp.pad(obj_vecs[sl], padrows).reshape(NPH, 128))
        pv.append(jnp.pad(pred_vecs[sl], padrows).reshape(NPH, 128))
        s_loc = triples[sl, :, 0]
        o_loc = triples[sl, :, 1]
        sidx.append(jnp.pad(s_loc, padidx).reshape(NPH))
        oidx.append(jnp.pad(o_loc, padidx).reshape(NPH))
        gs.append(sidx[hf] + jnp.broadcast_to(goff, (BH, OBJP)).reshape(NPH))
        go.append(oidx[hf] + jnp.broadcast_to(goff, (BH, OBJP)).reshape(NPH))
        counts8.append(None)

    for li, p in enumerate(params):
        Din = 128 if li == 0 else 256
        H = p['n1w1'].shape[1]
        Dout = p['n2w2'].shape[1]
        ws = p['n1w1'][:Din]
        wp = p['n1w1'][Din:2 * Din]
        wo = p['n1w1'][2 * Din:]
        b1 = p['n1b1'].reshape(1, H)
        w2s = p['n1w2'][:, :H]
        w2p = p['n1w2'][:, H:H + Dout]
        w2o = p['n1w2'][:, H + Dout:]
        b2s = p['n1b2'][:H].reshape(1, H)
        b2p = p['n1b2'][H:H + Dout].reshape(1, Dout)
        b2o = p['n1b2'][H + Dout:].reshape(1, H)

        cur, trip, pooled = {}, {}, {}
        for hf in range(NHALF):
            cur[hf] = _gather_call(Din)(ov[hf], gs[hf], go[hf])
        for hf in range(NHALF):
            trip[hf] = _triple_mlp_call(Din, H, Dout)(
                cur[hf][0], pv[hf], cur[hf][1],
                ws, wp, wo, b1, w2s, b2s, w2p, b2p, w2o, b2o)
        for hf in range(NHALF):
            nsa, npv, noa = trip[hf]
            if li == 0:
                pooled[hf], counts = _scatter_call(H, with_counts=True)(
                    nsa, noa, sidx[hf], oidx[hf])
                counts8[hf] = lax.slice(counts, (0, 0), (NPH, 8))
            else:
                pooled[hf] = _scatter_call(H)(nsa, noa, sidx[hf], oidx[hf])
            pv[hf] = npv
        for hf in range(NHALF):
            ov[hf] = _node_mlp_call(H, Dout)(
                pooled[hf], counts8[hf], p['n2w1'], p['n2b1'].reshape(1, H),
                p['n2w2'], p['n2b2'].reshape(1, Dout))

    outs = [o.reshape(BH, OBJP, -1)[:, :OBJ, :] for o in ov]
    return jnp.concatenate(outs, axis=0)


# speedup vs baseline: 4.5398x; 1.1494x over previous
import functools

import jax
import jax.numpy as jnp
from jax import lax
from jax.experimental import pallas as pl
from jax.experimental.pallas import tpu as pltpu
from jax.experimental.pallas import tpu_sc as plsc

B = 16
OBJ = 6250
NC, NS = 2, 16
NW = NC * NS
CH = 392
OBJP = NS * CH
NHALF = 2
BH = B // NHALF
NPH = BH * OBJP
BLK = 1792
CHG = 224


def _gather_call(D, dt=jnp.float32):
    mesh = plsc.VectorSubcoreMesh(core_axis_name="c", subcore_axis_name="s")
    sds = jax.ShapeDtypeStruct((NPH, D), dt)
    gpw = NPH // NW
    nchg = gpw // CHG

    @functools.partial(
        pl.kernel,
        out_type=(sds, sds),
        mesh=mesh,
        scratch_types=[
            pltpu.VMEM((CHG,), jnp.int32),
            pltpu.VMEM((CHG,), jnp.int32),
            pltpu.VMEM((CHG, D), dt),
            pltpu.VMEM((CHG, D), dt),
            pltpu.SemaphoreType.DMA,
            pltpu.SemaphoreType.DMA,
            pltpu.SemaphoreType.DMA,
            pltpu.SemaphoreType.DMA,
            pltpu.SemaphoreType.DMA,
        ],
    )
    def gather_k(table, gs, go, curs, curo,
                 idx0, idx1, rows0, rows1, gsem, is0, is1, ws0, ws1):
        c = lax.axis_index("c")
        t = lax.axis_index("s")
        start = (c * NS + t) * gpw
        idxb = (idx0, idx1)
        rowsb = (rows0, rows1)
        isem = (is0, is1)
        wsem = (ws0, ws1)
        tasks = []
        for idxarr, outref in ((gs, curs), (go, curo)):
            for j in range(nchg):
                tasks.append((idxarr, outref, start + j * CHG))
        nt = len(tasks)
        pend_idx = {}
        pend_out = {}

        def start_idx(k):
            idxarr, _, off = tasks[k]
            bi = k % 2
            pend_idx[k] = pltpu.async_copy(
                idxarr.at[pl.ds(off, CHG)], idxb[bi], isem[bi])

        start_idx(0)
        start_idx(1)
        for k in range(nt):
            bi = k % 2
            _, outref, off = tasks[k]
            if k >= 2:
                pend_out[k - 2].wait()
            pend_idx[k].wait()
            gd = pltpu.async_copy(table.at[idxb[bi]], rowsb[bi], gsem)
            if k + 2 < nt:
                start_idx(k + 2)
            gd.wait()
            pend_out[k] = pltpu.async_copy(
                rowsb[bi], outref.at[pl.ds(off, CHG)], wsem[bi])
        pend_out[nt - 2].wait()
        pend_out[nt - 1].wait()

    return gather_k


def _scatter_call(W, with_counts=False):
    mesh = plsc.VectorSubcoreMesh(core_axis_name="c", subcore_axis_name="s")

    npass = W // 128
    pooled_t = jax.ShapeDtypeStruct((NPH, W), jnp.float32)
    out_type = ((pooled_t, jax.ShapeDtypeStruct((NPH, 128), jnp.float32))
                if with_counts else pooled_t)
    SA = 200
    SB = CH - SA
    scratch = [
        pltpu.VMEM((SA, 128), jnp.float32),
        pltpu.VMEM((SB, 128), jnp.float32),
        pltpu.VMEM((56, 128), jnp.float32),
        pltpu.VMEM((SA,), jnp.int32),
        pltpu.VMEM((SB,), jnp.int32),
        pltpu.VMEM((SA,), jnp.int32),
        pltpu.VMEM((SB,), jnp.int32),
        pltpu.VMEM_SHARED((OBJP, 128), jnp.float32),
        pltpu.SemaphoreType.DMA,
        pltpu.SemaphoreType.DMA,
    ]

    @functools.partial(
        pl.kernel, out_type=out_type, mesh=mesh, scratch_types=scratch)
    def scatter_k(srcs, srco, sidx, oidx, *rest):
        if with_counts:
            (out, cnt, rows0, rows1, zb, idxsa, idxsb, idxoa, idxob, table,
             ls0, ls1) = rest
        else:
            (out, rows0, rows1, zb, idxsa, idxsb, idxoa, idxob, table,
             ls0, ls1) = rest
            cnt = None
        c = lax.axis_index("c")
        t = lax.axis_index("s")
        rowsb = (rows0, rows1)
        lsem = (ls0, ls1)
        zv = jnp.zeros((16,), jnp.float32)

        def fill(buf, lo, hi, vec):
            def frow(r, carry):
                for cc in range(128 // 16):
                    buf[r, pl.ds(cc * 16, 16)] = vec
                return carry
            lax.fori_loop(lo, hi, frow, 0)

        fill(zb, 0, 56, zv)

        def zero_table():
            for z in range(CH // 56):
                pltpu.sync_copy(zb, table.at[pl.ds(t * CH + z * 56, 56)])

        zero_table()

        nb = BH // NC
        loads = []
        idxrefs = []
        for j in range(nb):
            base = (c * nb + j) * OBJP + t * CH
            for cp in range(npass):
                loads.append((srcs, base, cp * 128, 0))
                loads.append((srcs, base + SA, cp * 128, 1))
                loads.append((srco, base, cp * 128, 0))
                loads.append((srco, base + SA, cp * 128, 1))
                idxrefs.extend([idxsa, idxsb, idxoa, idxob])

        def start_load(k):
            src, base, col, bi = loads[k]
            n = SA if bi == 0 else SB
            return pltpu.async_copy(
                src.at[pl.ds(base, n), pl.ds(col, 128)], rowsb[bi], lsem[bi])

        def load_idx(base):
            pltpu.sync_copy(sidx.at[pl.ds(base, SA)], idxsa)
            pltpu.sync_copy(sidx.at[pl.ds(base + SA, SB)], idxsb)
            pltpu.sync_copy(oidx.at[pl.ds(base, SA)], idxoa)
            pltpu.sync_copy(oidx.at[pl.ds(base + SA, SB)], idxob)

        pend = {0: start_load(0), 1: start_load(1)}
        k = 0
        for j in range(nb):
            base = (c * nb + j) * OBJP + t * CH
            load_idx(base)
            for cp in range(npass):
                col = cp * 128
                for _q in range(4):
                    pend.pop(k).wait()
                    pltpu.sync_copy(rowsb[k % 2], table.at[idxrefs[k]],
                                    add=True)
                    if k + 2 < len(loads):
                        pend[k + 2] = start_load(k + 2)
                    k += 1
                plsc.subcore_barrier()
                pltpu.sync_copy(table.at[pl.ds(t * CH, CH)],
                                out.at[pl.ds(base, CH), pl.ds(col, 128)])
                if k < len(loads) or with_counts:
                    zero_table()
                plsc.subcore_barrier()

        if with_counts:
            one = jnp.ones((16,), jnp.float32)
            fill(rows0, 0, SA, one)
            fill(rows1, 0, SB, one)

            @pl.when(t == NS - 1)
            def _():
                fill(rows1, OBJ - (NS - 1) * CH - SA, SB, zv)

            for j in range(nb):
                base = (c * nb + j) * OBJP + t * CH
                load_idx(base)
                pltpu.sync_copy(rows0, table.at[idxsa], add=True)
                pltpu.sync_copy(rows1, table.at[idxsb], add=True)
                pltpu.sync_copy(rows0, table.at[idxoa], add=True)
                pltpu.sync_copy(rows1, table.at[idxob], add=True)
                plsc.subcore_barrier()
                pltpu.sync_copy(table.at[pl.ds(t * CH, CH)],
                                cnt.at[pl.ds(base, CH)])
                if j + 1 < nb:
                    zero_table()
                plsc.subcore_barrier()

    return scatter_k


def _triple_mlp_call(Din, H, Dout, packed):
    grid = (NPH // BLK,)

    def unpack(x):
        u = lax.bitcast_convert_type(x, jnp.uint32)
        f0 = lax.bitcast_convert_type(jnp.left_shift(u, 16), jnp.float32)
        f1 = lax.bitcast_convert_type(u & jnp.uint32(0xFFFF0000),
                                      jnp.float32)
        return jnp.concatenate([f0, f1], axis=1)

    def body(s, p, o, ws, wp, wo, b1, w2s, b2s, w2p, b2p, w2o, b2o,
             nsa, npv, noa):
        m = pl.program_id(0)
        if packed:
            sf = unpack(s[...])
            of = unpack(o[...])
        else:
            sf = s[...]
            of = o[...]
        h = sf @ ws[...] + p[...] @ wp[...] + of @ wo[...] + b1[...]
        h = jnp.maximum(h, 0.0)
        rloc = (lax.broadcasted_iota(jnp.int32, (BLK, 1), 0) + m * BLK) % OBJP
        keep = (rloc < OBJ).astype(jnp.float32)
        nsa[...] = jnp.maximum(h @ w2s[...] + b2s[...], 0.0) * keep
        npv[...] = jnp.maximum(h @ w2p[...] + b2p[...], 0.0) * keep
        noa[...] = jnp.maximum(h @ w2o[...] + b2o[...], 0.0) * keep

    def full(shape):
        return pl.BlockSpec(shape, lambda m: (0, 0))

    rowblk = lambda d: pl.BlockSpec((BLK, d), lambda m: (m, 0))
    din_io = Din // 2 if packed else Din
    return pl.pallas_call(
        body,
        grid=grid,
        in_specs=[
            rowblk(din_io), rowblk(Din), rowblk(din_io),
            full((Din, H)), full((Din, H)), full((Din, H)), full((1, H)),
            full((H, H)), full((1, H)),
            full((H, Dout)), full((1, Dout)),
            full((H, H)), full((1, H)),
        ],
        out_specs=[rowblk(H), rowblk(Dout), rowblk(H)],
        out_shape=[
            jax.ShapeDtypeStruct((NPH, H), jnp.float32),
            jax.ShapeDtypeStruct((NPH, Dout), jnp.float32),
            jax.ShapeDtypeStruct((NPH, H), jnp.float32),
        ],
    )


def _node_mlp_call(H, Dout, pack_out=True):
    grid = (NPH // BLK,)

    def body(pa, cnt, w1, b1, w2, b2, outref):
        rcp = 1.0 / jnp.maximum(cnt[...][:, :1], 1.0)
        pooled = pa[...] * rcp
        h2 = jnp.maximum(pooled @ w1[...] + b1[...], 0.0)
        r = jnp.maximum(h2 @ w2[...] + b2[...], 0.0)
        if pack_out:
            u = lax.bitcast_convert_type(r, jnp.uint32)
            rnd = jnp.right_shift(
                u + jnp.uint32(0x7FFF) + (jnp.right_shift(u, 16) &
                                          jnp.uint32(1)), 16)
            lo = rnd[:, :Dout // 2]
            hi = rnd[:, Dout // 2:]
            outref[...] = lax.bitcast_convert_type(
                lo | jnp.left_shift(hi, 16), jnp.int32)
        else:
            outref[...] = r

    def full(shape):
        return pl.BlockSpec(shape, lambda m: (0, 0))

    return pl.pallas_call(
        body,
        grid=grid,
        in_specs=[
            pl.BlockSpec((BLK, H), lambda m: (m, 0)),
            pl.BlockSpec((BLK, 8), lambda m: (m, 0)),
            full((H, H)), full((1, H)), full((H, Dout)), full((1, Dout)),
        ],
        out_specs=pl.BlockSpec(
            (BLK, Dout // 2 if pack_out else Dout), lambda m: (m, 0)),
        out_shape=jax.ShapeDtypeStruct(
            (NPH, Dout // 2), jnp.int32) if pack_out else
        jax.ShapeDtypeStruct((NPH, Dout), jnp.float32),
    )


def kernel(obj_vecs, pred_vecs, triples, params):
    padrows = ((0, 0), (0, OBJP - OBJ), (0, 0))
    padidx = ((0, 0), (0, OBJP - OBJ))
    goff = (jnp.arange(BH, dtype=triples.dtype) * OBJP)[:, None]
    goff0 = (jnp.arange(BH, dtype=triples.dtype) * OBJ)[:, None]

    ov, pv, sidx, oidx, gs, go, gs0, go0, counts8 = ([] for _ in range(9))
    for hf in range(NHALF):
        sl = slice(hf * BH, (hf + 1) * BH)
        ov.append(obj_vecs[sl].reshape(BH * OBJ, 128))
        pv.append(jnp.pad(pred_vecs[sl], padrows).reshape(NPH, 128))
        s_loc = triples[sl, :, 0]
        o_loc = triples[sl, :, 1]
        sidx.append(jnp.pad(s_loc, padidx).reshape(NPH))
        oidx.append(jnp.pad(o_loc, padidx).reshape(NPH))
        gs.append(sidx[hf] + jnp.broadcast_to(goff, (BH, OBJP)).reshape(NPH))
        go.append(oidx[hf] + jnp.broadcast_to(goff, (BH, OBJP)).reshape(NPH))
        gs0.append(
            (jnp.pad(s_loc, padidx) + goff0).reshape(NPH))
        go0.append(
            (jnp.pad(o_loc, padidx) + goff0).reshape(NPH))
        counts8.append(None)

    for li, p in enumerate(params):
        Din = 128 if li == 0 else 256
        H = p['n1w1'].shape[1]
        Dout = p['n2w2'].shape[1]
        ws = p['n1w1'][:Din]
        wp = p['n1w1'][Din:2 * Din]
        wo = p['n1w1'][2 * Din:]
        b1 = p['n1b1'].reshape(1, H)
        w2s = p['n1w2'][:, :H]
        w2p = p['n1w2'][:, H:H + Dout]
        w2o = p['n1w2'][:, H + Dout:]
        b2s = p['n1b2'][:H].reshape(1, H)
        b2p = p['n1b2'][H:H + Dout].reshape(1, Dout)
        b2o = p['n1b2'][H + Dout:].reshape(1, H)

        cur, trip, pooled = {}, {}, {}
        for hf in range(NHALF):
            if li == 0:
                cur[hf] = _gather_call(Din)(ov[hf], gs0[hf], go0[hf])
            else:
                cur[hf] = _gather_call(Din // 2, jnp.int32)(
                    ov[hf], gs[hf], go[hf])
        for hf in range(NHALF):
            trip[hf] = _triple_mlp_call(Din, H, Dout, packed=li > 0)(
                cur[hf][0], pv[hf], cur[hf][1],
                ws, wp, wo, b1, w2s, b2s, w2p, b2p, w2o, b2o)
        for hf in range(NHALF):
            nsa, npv, noa = trip[hf]
            if li == 0:
                pooled[hf], counts = _scatter_call(H, with_counts=True)(
                    nsa, noa, sidx[hf], oidx[hf])
                counts8[hf] = lax.slice(counts, (0, 0), (NPH, 8))
            else:
                pooled[hf] = _scatter_call(H)(nsa, noa, sidx[hf], oidx[hf])
            pv[hf] = npv
        pack_out = li < len(params) - 1
        for hf in range(NHALF):
            ov[hf] = _node_mlp_call(H, Dout, pack_out)(
                pooled[hf], counts8[hf], p['n2w1'], p['n2b1'].reshape(1, H),
                p['n2w2'], p['n2b2'].reshape(1, Dout))

    outs = [o.reshape(BH, OBJP, -1)[:, :OBJ, :] for o in ov]
    return jnp.concatenate(outs, axis=0)
